# baseline (speedup 1.0000x reference)
import functools

import jax
import jax.numpy as jnp
from jax.experimental import pallas as pl
from jax.experimental.pallas import tpu as pltpu
from jax.experimental.pallas import tpu_sc as plsc

B = 4096
L = 50
D_EMB = 64
N_TYPES = 9
LN_EPS = 1e-3

_NC = 2
_NS = 16
_NW = _NC * _NS
_B_BLK = 256
_G = 128


def _sc_gather_all(pair_tab, tok_idx, handle_idx, type_table, type_idx):
    n_tok = tok_idx.size
    tok_pw = n_tok // _NW
    n_ch = tok_pw // _G
    u_pw = B // _NW
    d2 = pair_tab.shape[1]
    d_t = type_table.shape[1]
    tok3 = tok_idx.reshape(_NW, n_ch, _G)
    hand2 = handle_idx.reshape(_NW, u_pw)
    typ2 = type_idx.reshape(_NW, u_pw)
    mesh = plsc.VectorSubcoreMesh(core_axis_name="c", subcore_axis_name="s")

    @functools.partial(
        pl.kernel, mesh=mesh,
        out_type=(jax.ShapeDtypeStruct((n_tok, d2), jnp.float32),
                  jax.ShapeDtypeStruct((B, d2), jnp.float32),
                  jax.ShapeDtypeStruct((B, d_t), jnp.float32)),
        scratch_types=[pltpu.VMEM((n_ch, _G), jnp.int32),
                       pltpu.VMEM((_G, d2), jnp.float32),
                       pltpu.VMEM((_G, d2), jnp.float32),
                       pltpu.VMEM((u_pw,), jnp.int32),
                       pltpu.VMEM((u_pw, d2), jnp.float32),
                       pltpu.VMEM((u_pw, d_t), jnp.float32),
                       pltpu.SemaphoreType.DMA,
                       pltpu.SemaphoreType.DMA])
    def gather_kernel(ptab, cidx, uidx, ttab, tidx, cout, uout, tout,
                      cidx_v, rows_a, rows_b, sidx_v, urows_v, trows_v,
                      sem_a, sem_b):
        wid = jax.lax.axis_index("s") * _NC + jax.lax.axis_index("c")
        cbase = wid * tok_pw
        pltpu.sync_copy(cidx.at[wid], cidx_v)

        @pl.loop(0, n_ch, step=2)
        def _(c):
            ca = pltpu.async_copy(ptab.at[cidx_v.at[c]], rows_a, sem_a)
            cb = pltpu.async_copy(ptab.at[cidx_v.at[c + 1]], rows_b, sem_b)
            ca.wait()
            pltpu.sync_copy(rows_a, cout.at[pl.ds(cbase + c * _G, _G)])
            cb.wait()
            pltpu.sync_copy(rows_b, cout.at[pl.ds(cbase + (c + 1) * _G, _G)])

        ubase = wid * u_pw
        pltpu.sync_copy(uidx.at[wid], sidx_v)
        pltpu.async_copy(ptab.at[sidx_v], urows_v, sem_a).wait()
        pltpu.sync_copy(urows_v, uout.at[pl.ds(ubase, u_pw)])

        pltpu.sync_copy(tidx.at[wid], sidx_v)
        pltpu.async_copy(ttab.at[sidx_v], trows_v, sem_a).wait()
        pltpu.sync_copy(trows_v, tout.at[pl.ds(ubase, u_pw)])

    return gather_kernel(pair_tab, tok3, hand2, type_table, typ2)


def _tc_mlp_kernel(cont_ref, par_ref, vis_ref, vpm_ref, typeb_ref, w1a_ref,
                   w1c_ref, w2_ref, b2_ref, g_ref, beta_ref, wout_ref,
                   out_ref):
    x = cont_ref[...]
    par = par_ref[...]
    acc_all = jnp.zeros((x.shape[1], x.shape[2]), jnp.float32)
    acc_even = jnp.zeros((x.shape[1], x.shape[2]), jnp.float32)
    for t in range(L):
        x_t = x[t]
        acc_all += x_t
        acc_even += x_t * (1.0 - par[:, t:t + 1])
    acc_odd = acc_all - acc_even
    qm = (acc_even[:, :D_EMB] + acc_odd[:, D_EMB:]) * (1.0 / L)

    vg = vis_ref[...]
    vpm = vpm_ref[...]
    vis = vg[:, :D_EMB] * (1.0 - vpm) + vg[:, D_EMB:] * vpm

    h1 = jnp.dot(qm, w1a_ref[...], preferred_element_type=jnp.float32)
    h1 += jnp.dot(vis, w1c_ref[...], preferred_element_type=jnp.float32)
    h1 += typeb_ref[...]
    h1 = jnp.maximum(h1, 0.0)
    h2 = jnp.dot(h1, w2_ref[...], preferred_element_type=jnp.float32)
    h2 += b2_ref[...]
    h2 = jnp.maximum(h2, 0.0)
    mu = jnp.mean(h2, axis=-1, keepdims=True)
    dev = h2 - mu
    var = jnp.mean(dev * dev, axis=-1, keepdims=True)
    hn = dev * jax.lax.rsqrt(var + LN_EPS) * g_ref[...] + beta_ref[...]
    out_ref[...] = jnp.dot(hn, wout_ref[...], preferred_element_type=jnp.float32)


def kernel(content_tokens, user_type_idx, user_handle_idx, content_table,
           user_table, W1, b1, W2, b2, ln_gamma, ln_beta, W_out):
    n_out = W_out.shape[1]
    type_table = W1[D_EMB:D_EMB + N_TYPES] + b1[None, :]
    v_c = content_table.shape[0]
    v_u = user_table.shape[0]
    stack = jnp.concatenate(
        [content_table, user_table, jnp.zeros((1, D_EMB), jnp.float32)],
        axis=0)
    pair_tab = stack.reshape((v_c + v_u + 1) // 2, 2 * D_EMB)

    tok_tm = content_tokens.T.reshape(B * L)
    tok_pair = tok_tm // 2
    par_tc = (content_tokens & 1).astype(jnp.float32)
    hand_pair = v_c // 2 + user_handle_idx // 2
    vis_par = jnp.broadcast_to(
        (user_handle_idx & 1).astype(jnp.float32)[:, None], (B, D_EMB))

    cont, vis, typeb = _sc_gather_all(
        pair_tab, tok_pair, hand_pair, type_table, user_type_idx)

    cont3 = cont.reshape(L, B, 2 * D_EMB)
    w1a = W1[:D_EMB]
    w1c = W1[D_EMB + N_TYPES:]

    grid = (B // _B_BLK,)
    return pl.pallas_call(
        _tc_mlp_kernel,
        grid=grid,
        in_specs=[
            pl.BlockSpec((L, _B_BLK, 2 * D_EMB), lambda i: (0, i, 0)),
            pl.BlockSpec((_B_BLK, L), lambda i: (i, 0)),
            pl.BlockSpec((_B_BLK, 2 * D_EMB), lambda i: (i, 0)),
            pl.BlockSpec((_B_BLK, D_EMB), lambda i: (i, 0)),
            pl.BlockSpec((_B_BLK, W1.shape[1]), lambda i: (i, 0)),
            pl.BlockSpec(w1a.shape, lambda i: (0, 0)),
            pl.BlockSpec(w1c.shape, lambda i: (0, 0)),
            pl.BlockSpec(W2.shape, lambda i: (0, 0)),
            pl.BlockSpec((1, W2.shape[1]), lambda i: (0, 0)),
            pl.BlockSpec((1, W2.shape[1]), lambda i: (0, 0)),
            pl.BlockSpec((1, W2.shape[1]), lambda i: (0, 0)),
            pl.BlockSpec(W_out.shape, lambda i: (0, 0)),
        ],
        out_specs=pl.BlockSpec((_B_BLK, n_out), lambda i: (i, 0)),
        out_shape=jax.ShapeDtypeStruct((B, n_out), jnp.float32),
    )(cont3, par_tc, vis, vis_par, typeb, w1a, w1c, W2, b2.reshape(1, -1),
      ln_gamma.reshape(1, -1), ln_beta.reshape(1, -1), W_out)


# speedup vs baseline: 1.3891x; 1.3891x over previous
import functools

import jax
import jax.numpy as jnp
from jax.experimental import pallas as pl
from jax.experimental.pallas import tpu as pltpu
from jax.experimental.pallas import tpu_sc as plsc

B = 4096
L = 50
D_EMB = 64
N_TYPES = 9
LN_EPS = 1e-3

_NC = 2
_NS = 16
_NW = _NC * _NS
_B_BLK = 256
_G = 128
_WIDEN_BLK = 4096


def _dup_kernel(in_ref, out_ref):
    x = in_ref[...]
    out_ref[:, :D_EMB] = x
    out_ref[:, D_EMB:] = x


def _widen(table):
    v = table.shape[0]
    g = (v + _WIDEN_BLK - 1) // _WIDEN_BLK
    return pl.pallas_call(
        _dup_kernel, grid=(g,),
        in_specs=[pl.BlockSpec((_WIDEN_BLK, D_EMB), lambda i: (i, 0))],
        out_specs=pl.BlockSpec((_WIDEN_BLK, 2 * D_EMB), lambda i: (i, 0)),
        out_shape=jax.ShapeDtypeStruct((v, 2 * D_EMB), jnp.float32),
    )(table)


def _sc_gather_all(content_tab2, tok_idx, user_tab2, handle_idx,
                   type_table, type_idx):
    n_tok = tok_idx.size
    tok_pw = n_tok // _NW
    n_ch = tok_pw // _G
    u_pw = B // _NW
    d2 = content_tab2.shape[1]
    d_t = type_table.shape[1]
    tok3 = tok_idx.reshape(_NW, n_ch, _G)
    hand2 = handle_idx.reshape(_NW, u_pw)
    typ2 = type_idx.reshape(_NW, u_pw)
    mesh = plsc.VectorSubcoreMesh(core_axis_name="c", subcore_axis_name="s")

    @functools.partial(
        pl.kernel, mesh=mesh,
        out_type=(jax.ShapeDtypeStruct((n_tok, d2), jnp.float32),
                  jax.ShapeDtypeStruct((B, d2), jnp.float32),
                  jax.ShapeDtypeStruct((B, d_t), jnp.float32)),
        scratch_types=[pltpu.VMEM((n_ch, _G), jnp.int32),
                       pltpu.VMEM((_G, d2), jnp.float32),
                       pltpu.VMEM((_G, d2), jnp.float32),
                       pltpu.VMEM((u_pw,), jnp.int32),
                       pltpu.VMEM((u_pw, d2), jnp.float32),
                       pltpu.VMEM((u_pw, d_t), jnp.float32),
                       pltpu.SemaphoreType.DMA,
                       pltpu.SemaphoreType.DMA])
    def gather_kernel(ctab, cidx, utab, uidx, ttab, tidx, cout, uout, tout,
                      cidx_v, rows_a, rows_b, sidx_v, urows_v, trows_v,
                      sem_a, sem_b):
        wid = jax.lax.axis_index("s") * _NC + jax.lax.axis_index("c")
        cbase = wid * tok_pw
        pltpu.sync_copy(cidx.at[wid], cidx_v)

        @pl.loop(0, n_ch, step=2)
        def _(c):
            ca = pltpu.async_copy(ctab.at[cidx_v.at[c]], rows_a, sem_a)
            cb = pltpu.async_copy(ctab.at[cidx_v.at[c + 1]], rows_b, sem_b)
            ca.wait()
            pltpu.sync_copy(rows_a, cout.at[pl.ds(cbase + c * _G, _G)])
            cb.wait()
            pltpu.sync_copy(rows_b, cout.at[pl.ds(cbase + (c + 1) * _G, _G)])

        ubase = wid * u_pw
        pltpu.sync_copy(uidx.at[wid], sidx_v)
        pltpu.async_copy(utab.at[sidx_v], urows_v, sem_a).wait()
        pltpu.sync_copy(urows_v, uout.at[pl.ds(ubase, u_pw)])

        pltpu.sync_copy(tidx.at[wid], sidx_v)
        pltpu.async_copy(ttab.at[sidx_v], trows_v, sem_a).wait()
        pltpu.sync_copy(trows_v, tout.at[pl.ds(ubase, u_pw)])

    return gather_kernel(content_tab2, tok3, user_tab2, hand2,
                         type_table, typ2)


def _tc_mlp_kernel(cont_ref, vis_ref, typeb_ref, w1a_ref, w1c_ref, w2_ref,
                   b2_ref, g_ref, beta_ref, wout_ref, out_ref):
    x = cont_ref[...]
    qm = jnp.mean(x, axis=0)[:, :D_EMB]
    h1 = jnp.dot(qm, w1a_ref[...], preferred_element_type=jnp.float32)
    h1 += jnp.dot(vis_ref[...][:, :D_EMB], w1c_ref[...],
                  preferred_element_type=jnp.float32)
    h1 += typeb_ref[...]
    h1 = jnp.maximum(h1, 0.0)
    h2 = jnp.dot(h1, w2_ref[...], preferred_element_type=jnp.float32)
    h2 += b2_ref[...]
    h2 = jnp.maximum(h2, 0.0)
    mu = jnp.mean(h2, axis=-1, keepdims=True)
    dev = h2 - mu
    var = jnp.mean(dev * dev, axis=-1, keepdims=True)
    hn = dev * jax.lax.rsqrt(var + LN_EPS) * g_ref[...] + beta_ref[...]
    out_ref[...] = jnp.dot(hn, wout_ref[...], preferred_element_type=jnp.float32)


def kernel(content_tokens, user_type_idx, user_handle_idx, content_table,
           user_table, W1, b1, W2, b2, ln_gamma, ln_beta, W_out):
    n_out = W_out.shape[1]
    type_table = W1[D_EMB:D_EMB + N_TYPES] + b1[None, :]
    ctab2 = _widen(content_table)
    utab2 = _widen(user_table)
    cont, vis, typeb = _sc_gather_all(
        ctab2, content_tokens.T.reshape(B * L), utab2,
        user_handle_idx, type_table, user_type_idx)

    cont3 = cont.reshape(L, B, 2 * D_EMB)
    w1a = W1[:D_EMB]
    w1c = W1[D_EMB + N_TYPES:]

    grid = (B // _B_BLK,)
    return pl.pallas_call(
        _tc_mlp_kernel,
        grid=grid,
        in_specs=[
            pl.BlockSpec((L, _B_BLK, 2 * D_EMB), lambda i: (0, i, 0)),
            pl.BlockSpec((_B_BLK, 2 * D_EMB), lambda i: (i, 0)),
            pl.BlockSpec((_B_BLK, W1.shape[1]), lambda i: (i, 0)),
            pl.BlockSpec(w1a.shape, lambda i: (0, 0)),
            pl.BlockSpec(w1c.shape, lambda i: (0, 0)),
            pl.BlockSpec(W2.shape, lambda i: (0, 0)),
            pl.BlockSpec((1, W2.shape[1]), lambda i: (0, 0)),
            pl.BlockSpec((1, W2.shape[1]), lambda i: (0, 0)),
            pl.BlockSpec((1, W2.shape[1]), lambda i: (0, 0)),
            pl.BlockSpec(W_out.shape, lambda i: (0, 0)),
        ],
        out_specs=pl.BlockSpec((_B_BLK, n_out), lambda i: (i, 0)),
        out_shape=jax.ShapeDtypeStruct((B, n_out), jnp.float32),
    )(cont3, vis, typeb, w1a, w1c, W2, b2.reshape(1, -1),
      ln_gamma.reshape(1, -1), ln_beta.reshape(1, -1), W_out)


# speedup vs baseline: 1.4578x; 1.0495x over previous
import functools

import jax
import jax.numpy as jnp
from jax.experimental import pallas as pl
from jax.experimental.pallas import tpu as pltpu
from jax.experimental.pallas import tpu_sc as plsc

B = 4096
L = 50
D_EMB = 64
N_TYPES = 9
LN_EPS = 1e-3

_NC = 2
_NS = 16
_NW = _NC * _NS
_B_BLK = 256
_G = 128
_WIDEN_BLK = 4096


def _dup_kernel(in_ref, out_ref):
    x = in_ref[...]
    out_ref[:, :D_EMB] = x
    out_ref[:, D_EMB:] = x


def _widen(table):
    v = table.shape[0]
    g = (v + _WIDEN_BLK - 1) // _WIDEN_BLK
    return pl.pallas_call(
        _dup_kernel, grid=(g,),
        in_specs=[pl.BlockSpec((_WIDEN_BLK, D_EMB), lambda i: (i, 0))],
        out_specs=pl.BlockSpec((_WIDEN_BLK, 2 * D_EMB), lambda i: (i, 0)),
        out_shape=jax.ShapeDtypeStruct((v, 2 * D_EMB), jnp.float32),
    )(table)


def _sc_gather_all(content_tab2, tok_idx, user_tab2, handle_idx,
                   type_table, type_idx):
    n_tok = tok_idx.size
    tok_pw = n_tok // _NW
    n_ch = tok_pw // _G
    u_pw = B // _NW
    d2 = content_tab2.shape[1]
    d_t = type_table.shape[1]
    tok3 = tok_idx.reshape(_NW, n_ch, _G)
    hand2 = handle_idx.reshape(_NW, u_pw)
    typ2 = type_idx.reshape(_NW, u_pw)
    mesh = plsc.VectorSubcoreMesh(core_axis_name="c", subcore_axis_name="s")

    @functools.partial(
        pl.kernel, mesh=mesh,
        out_type=(jax.ShapeDtypeStruct((n_tok, d2), jnp.float32),
                  jax.ShapeDtypeStruct((B, d2), jnp.float32),
                  jax.ShapeDtypeStruct((B, d_t), jnp.float32)),
        scratch_types=[pltpu.VMEM((n_ch, _G), jnp.int32),
                       pltpu.VMEM((_G, d2), jnp.float32),
                       pltpu.VMEM((_G, d2), jnp.float32),
                       pltpu.VMEM((u_pw,), jnp.int32),
                       pltpu.VMEM((u_pw, d2), jnp.float32),
                       pltpu.VMEM((u_pw, d_t), jnp.float32),
                       pltpu.SemaphoreType.DMA,
                       pltpu.SemaphoreType.DMA])
    def gather_kernel(ctab, cidx, utab, uidx, ttab, tidx, cout, uout, tout,
                      cidx_v, rows_a, rows_b, sidx_v, urows_v, trows_v,
                      sem_a, sem_b):
        wid = jax.lax.axis_index("s") * _NC + jax.lax.axis_index("c")
        cbase = wid * tok_pw
        pltpu.sync_copy(cidx.at[wid], cidx_v)

        @pl.loop(0, n_ch, step=2)
        def _(c):
            ca = pltpu.async_copy(ctab.at[cidx_v.at[c]], rows_a, sem_a)
            cb = pltpu.async_copy(ctab.at[cidx_v.at[c + 1]], rows_b, sem_b)
            ca.wait()
            pltpu.sync_copy(rows_a, cout.at[pl.ds(cbase + c * _G, _G)])
            cb.wait()
            pltpu.sync_copy(rows_b, cout.at[pl.ds(cbase + (c + 1) * _G, _G)])

        ubase = wid * u_pw
        pltpu.sync_copy(uidx.at[wid], sidx_v)
        pltpu.async_copy(utab.at[sidx_v], urows_v, sem_a).wait()
        pltpu.sync_copy(urows_v, uout.at[pl.ds(ubase, u_pw)])

        pltpu.sync_copy(tidx.at[wid], sidx_v)
        pltpu.async_copy(ttab.at[sidx_v], trows_v, sem_a).wait()
        pltpu.sync_copy(trows_v, tout.at[pl.ds(ubase, u_pw)])

    return gather_kernel(content_tab2, tok3, user_tab2, hand2,
                         type_table, typ2)


def _tc_mlp_kernel(cont_ref, vis_ref, typeb_ref, w1a_ref, w1c_ref, w2_ref,
                   b2_ref, g_ref, beta_ref, wout_ref, out_ref):
    x = cont_ref[...]
    qm = jnp.mean(x, axis=0)[:, :D_EMB]
    h1 = jnp.dot(qm, w1a_ref[...], preferred_element_type=jnp.float32)
    h1 += jnp.dot(vis_ref[...][:, :D_EMB], w1c_ref[...],
                  preferred_element_type=jnp.float32)
    h1 += typeb_ref[...]
    h1 = jnp.maximum(h1, 0.0)
    h2 = jnp.dot(h1, w2_ref[...], preferred_element_type=jnp.float32)
    h2 += b2_ref[...]
    h2 = jnp.maximum(h2, 0.0)
    mu = jnp.mean(h2, axis=-1, keepdims=True)
    dev = h2 - mu
    var = jnp.mean(dev * dev, axis=-1, keepdims=True)
    hn = dev * jax.lax.rsqrt(var + LN_EPS) * g_ref[...] + beta_ref[...]
    out_ref[...] = jnp.dot(hn, wout_ref[...], preferred_element_type=jnp.float32)


def kernel(content_tokens, user_type_idx, user_handle_idx, content_table,
           user_table, W1, b1, W2, b2, ln_gamma, ln_beta, W_out):
    n_out = W_out.shape[1]
    type_table = W1[D_EMB:D_EMB + N_TYPES] + b1[None, :]
    ctab2 = jnp.concatenate([content_table, content_table], axis=1)
    utab2 = jnp.concatenate([user_table, user_table], axis=1)
    cont, vis, typeb = _sc_gather_all(
        ctab2, content_tokens.T.reshape(B * L), utab2,
        user_handle_idx, type_table, user_type_idx)

    cont3 = cont.reshape(L, B, 2 * D_EMB)
    w1a = W1[:D_EMB]
    w1c = W1[D_EMB + N_TYPES:]

    grid = (B // _B_BLK,)
    return pl.pallas_call(
        _tc_mlp_kernel,
        grid=grid,
        in_specs=[
            pl.BlockSpec((L, _B_BLK, 2 * D_EMB), lambda i: (0, i, 0)),
            pl.BlockSpec((_B_BLK, 2 * D_EMB), lambda i: (i, 0)),
            pl.BlockSpec((_B_BLK, W1.shape[1]), lambda i: (i, 0)),
            pl.BlockSpec(w1a.shape, lambda i: (0, 0)),
            pl.BlockSpec(w1c.shape, lambda i: (0, 0)),
            pl.BlockSpec(W2.shape, lambda i: (0, 0)),
            pl.BlockSpec((1, W2.shape[1]), lambda i: (0, 0)),
            pl.BlockSpec((1, W2.shape[1]), lambda i: (0, 0)),
            pl.BlockSpec((1, W2.shape[1]), lambda i: (0, 0)),
            pl.BlockSpec(W_out.shape, lambda i: (0, 0)),
        ],
        out_specs=pl.BlockSpec((_B_BLK, n_out), lambda i: (i, 0)),
        out_shape=jax.ShapeDtypeStruct((B, n_out), jnp.float32),
    )(cont3, vis, typeb, w1a, w1c, W2, b2.reshape(1, -1),
      ln_gamma.reshape(1, -1), ln_beta.reshape(1, -1), W_out)
